# baseline (speedup 1.0000x reference)
import jax
import jax.numpy as jnp
from jax.experimental import pallas as pl
from jax.experimental.pallas import tpu as pltpu


def _gemm_body(x_ref, w_ref, o_ref):
    o_ref[...] = jnp.dot(
        w_ref[...], x_ref[...],
        preferred_element_type=jnp.float32).astype(jnp.bfloat16)


def kernel(x, w_element, w_restore):
    N, Cin, H, W = x.shape
    Cout = w_restore.shape[0]
    HW = H * W
    w1 = w_element[:, :, 0, 0].astype(jnp.float32)
    w2 = w_restore[:, :, 0, 0].astype(jnp.float32)
    wf = jnp.dot(w2, w1).astype(jnp.bfloat16)

    x_bf = x.reshape(N, Cin, HW).astype(jnp.bfloat16)

    out_bf = pl.pallas_call(
        _gemm_body,
        out_shape=jax.ShapeDtypeStruct((N, Cout, HW), jnp.bfloat16),
        grid=(N,),
        in_specs=[pl.BlockSpec((None, Cin, HW), lambda n: (n, 0, 0)),
                  pl.BlockSpec((Cout, Cin), lambda n: (0, 0))],
        out_specs=pl.BlockSpec((None, Cout, HW), lambda n: (n, 0, 0)),
        compiler_params=pltpu.CompilerParams(
            dimension_semantics=("parallel",),
            vmem_limit_bytes=40 << 20),
        cost_estimate=pl.CostEstimate(
            flops=2 * N * HW * Cin * Cout, transcendentals=0,
            bytes_accessed=N * HW * (Cin + Cout) * 2),
    )(x_bf, wf)
    return out_bf.astype(jnp.float32).reshape(N, Cout, H, W)
